# Initial kernel scaffold; baseline (speedup 1.0000x reference)
import functools

import jax
import jax.numpy as jnp
from jax import lax
from jax.experimental import pallas as pl
from jax.experimental.pallas import tpu as pltpu
from jax.experimental.pallas import tpu_sc as plsc

DA = 144
CH = 128
NROWS_PAD = 10240


def _sc_aggregate(xa, colp, rowp, n_chunks_per_worker):
    info = plsc.get_sparse_core_info()
    nc, ns = info.num_cores, info.num_subcores
    rows_per_tile = NROWS_PAD // ns
    mesh = plsc.VectorSubcoreMesh(core_axis_name="c", subcore_axis_name="s")
    kpw = n_chunks_per_worker

    @functools.partial(
        pl.kernel,
        mesh=mesh,
        out_type=jax.ShapeDtypeStruct((nc, NROWS_PAD, DA), jnp.float32),
        scratch_types=[
            pltpu.VMEM((CH,), jnp.int32),
            pltpu.VMEM((CH,), jnp.int32),
            pltpu.VMEM((CH, DA), jnp.float32),
            pltpu.VMEM_SHARED((NROWS_PAD, DA), jnp.float32),
            pltpu.SemaphoreType.DMA,
        ],
    )
    def k(xa_hbm, col_hbm, row_hbm, out_hbm, colc, rowc, rows, agg, sem):
        c = lax.axis_index("c")
        s = lax.axis_index("s")
        wid = c * ns + s

        def zrow(i, carry):
            for j in range(DA // 16):
                rows[i, pl.ds(j * 16, 16)] = jnp.zeros((16,), jnp.float32)
            return carry

        lax.fori_loop(0, CH, zrow, 0)
        for t in range(rows_per_tile // CH):
            pltpu.sync_copy(rows, agg.at[pl.ds(s * rows_per_tile + t * CH, CH)])
        plsc.subcore_barrier()

        base_e = wid * kpw * CH

        def body(g, carry):
            e0 = base_e + g * CH
            pltpu.sync_copy(col_hbm.at[pl.ds(e0, CH)], colc)
            pltpu.sync_copy(row_hbm.at[pl.ds(e0, CH)], rowc)
            pltpu.async_copy(xa_hbm.at[colc], rows, sem).wait()
            pltpu.sync_copy(rows, agg.at[rowc], add=True)
            return carry

        lax.fori_loop(0, kpw, body, 0)
        plsc.subcore_barrier()
        pltpu.sync_copy(
            agg.at[pl.ds(s * rows_per_tile, rows_per_tile)],
            out_hbm.at[c, pl.ds(s * rows_per_tile, rows_per_tile)],
        )

    return k(xa, colp, rowp)


def _tc_finish_body(xa_ref, p0_ref, p1_ref, wa_ref, g_ref, b_ref, o_ref):
    s = xa_ref[...] + p0_ref[...] + p1_ref[...]
    h = lax.dot_general(
        s, wa_ref[...], (((1,), (1,)), ((), ())),
        preferred_element_type=jnp.float32,
    )
    mean = jnp.mean(h, axis=1, keepdims=True)
    d = h - mean
    var = jnp.mean(d * d, axis=1, keepdims=True)
    y = d * lax.rsqrt(var + 1e-5) * g_ref[...] + b_ref[...]
    o_ref[...] = jnp.maximum(y, 0.0)


def _tc_finish(xa, p0, p1, wa, gamma2, beta2):
    n = xa.shape[0]
    dout = wa.shape[0]
    bs = 2000
    grid = n // bs
    return pl.pallas_call(
        _tc_finish_body,
        grid=(grid,),
        in_specs=[
            pl.BlockSpec((bs, DA), lambda i: (i, 0)),
            pl.BlockSpec((bs, DA), lambda i: (i, 0)),
            pl.BlockSpec((bs, DA), lambda i: (i, 0)),
            pl.BlockSpec((dout, DA), lambda i: (0, 0)),
            pl.BlockSpec((1, dout), lambda i: (0, 0)),
            pl.BlockSpec((1, dout), lambda i: (0, 0)),
        ],
        out_specs=pl.BlockSpec((bs, dout), lambda i: (i, 0)),
        out_shape=jax.ShapeDtypeStruct((n, dout), jnp.float32),
    )(xa, p0, p1, wa, gamma2, beta2)


def kernel(x, edge_index, W, b, gamma, beta):
    n, d_in = x.shape
    d_out = W.shape[0]
    e = edge_index.shape[1]
    row = edge_index[0].astype(jnp.int32)
    col = edge_index[1].astype(jnp.int32)

    xa = jnp.concatenate(
        [x, jnp.ones((n, 1), jnp.float32), jnp.zeros((n, DA - d_in - 1), jnp.float32)],
        axis=1,
    )
    wa = jnp.concatenate(
        [W, b[:, None], jnp.zeros((d_out, DA - d_in - 1), jnp.float32)], axis=1
    )

    info = plsc.get_sparse_core_info()
    nw = info.num_cores * info.num_subcores
    kpw = -(-e // (nw * CH))
    e_pad = kpw * nw * CH
    trash = n
    colp = jnp.concatenate([col, jnp.zeros((e_pad - e,), jnp.int32)])
    rowp = jnp.concatenate([row, jnp.full((e_pad - e,), trash, jnp.int32)])

    parts = _sc_aggregate(xa, colp, rowp, kpw)
    p0 = parts[0, :n]
    p1 = parts[1, :n]
    return _tc_finish(xa, p0, p1, wa, gamma.reshape(1, d_out), beta.reshape(1, d_out))


# speedup vs baseline: 3.5798x; 3.5798x over previous
import functools

import jax
import jax.numpy as jnp
from jax import lax
from jax.experimental import pallas as pl
from jax.experimental.pallas import tpu as pltpu
from jax.experimental.pallas import tpu_sc as plsc

DA = 144
CH = 128
NROWS_PAD = 10240


def _sc_aggregate(xa, colp, rowp, n_chunks_per_worker):
    info = plsc.get_sparse_core_info()
    nc, ns = info.num_cores, info.num_subcores
    rows_per_tile = NROWS_PAD // ns
    mesh = plsc.VectorSubcoreMesh(core_axis_name="c", subcore_axis_name="s")
    kpw = n_chunks_per_worker

    @functools.partial(
        pl.kernel,
        mesh=mesh,
        compiler_params=pltpu.CompilerParams(use_tc_tiling_on_sc=False),
        out_type=jax.ShapeDtypeStruct((nc, NROWS_PAD, DA), jnp.float32),
        scratch_types=[
            pltpu.VMEM((CH,), jnp.int32),
            pltpu.VMEM((CH,), jnp.int32),
            pltpu.VMEM((CH, DA), jnp.float32),
            pltpu.VMEM_SHARED((NROWS_PAD, DA), jnp.float32),
            pltpu.SemaphoreType.DMA,
        ],
    )
    def k(xa_hbm, col_hbm, row_hbm, out_hbm, colc, rowc, rows, agg, sem):
        c = lax.axis_index("c")
        s = lax.axis_index("s")
        wid = c * ns + s

        def zrow(i, carry):
            for j in range(DA // 16):
                rows[i, pl.ds(j * 16, 16)] = jnp.zeros((16,), jnp.float32)
            return carry

        lax.fori_loop(0, CH, zrow, 0)
        for t in range(rows_per_tile // CH):
            pltpu.sync_copy(rows, agg.at[pl.ds(s * rows_per_tile + t * CH, CH)])
        plsc.subcore_barrier()

        base_e = wid * kpw * CH

        def body(g, carry):
            e0 = base_e + g * CH
            pltpu.sync_copy(col_hbm.at[pl.ds(e0, CH)], colc)
            pltpu.sync_copy(row_hbm.at[pl.ds(e0, CH)], rowc)
            pltpu.async_copy(xa_hbm.at[colc], rows, sem).wait()
            pltpu.sync_copy(rows, agg.at[rowc], add=True)
            return carry

        lax.fori_loop(0, kpw, body, 0)
        plsc.subcore_barrier()
        pltpu.sync_copy(
            agg.at[pl.ds(s * rows_per_tile, rows_per_tile)],
            out_hbm.at[c, pl.ds(s * rows_per_tile, rows_per_tile)],
        )

    return k(xa, colp, rowp)


def _tc_finish_body(xa_ref, p0_ref, p1_ref, wa_ref, g_ref, b_ref, o_ref):
    s = xa_ref[...] + p0_ref[...] + p1_ref[...]
    h = lax.dot_general(
        s, wa_ref[...], (((1,), (1,)), ((), ())),
        preferred_element_type=jnp.float32,
    )
    mean = jnp.mean(h, axis=1, keepdims=True)
    d = h - mean
    var = jnp.mean(d * d, axis=1, keepdims=True)
    y = d * lax.rsqrt(var + 1e-5) * g_ref[...] + b_ref[...]
    o_ref[...] = jnp.maximum(y, 0.0)


def _tc_finish(xa, p0, p1, wa, gamma2, beta2):
    n = xa.shape[0]
    dout = wa.shape[0]
    bs = 2000
    grid = n // bs
    return pl.pallas_call(
        _tc_finish_body,
        grid=(grid,),
        in_specs=[
            pl.BlockSpec((bs, DA), lambda i: (i, 0)),
            pl.BlockSpec((bs, DA), lambda i: (i, 0)),
            pl.BlockSpec((bs, DA), lambda i: (i, 0)),
            pl.BlockSpec((dout, DA), lambda i: (0, 0)),
            pl.BlockSpec((1, dout), lambda i: (0, 0)),
            pl.BlockSpec((1, dout), lambda i: (0, 0)),
        ],
        out_specs=pl.BlockSpec((bs, dout), lambda i: (i, 0)),
        out_shape=jax.ShapeDtypeStruct((n, dout), jnp.float32),
    )(xa, p0, p1, wa, gamma2, beta2)


def kernel(x, edge_index, W, b, gamma, beta):
    n, d_in = x.shape
    d_out = W.shape[0]
    e = edge_index.shape[1]
    row = edge_index[0].astype(jnp.int32)
    col = edge_index[1].astype(jnp.int32)

    xa = jnp.concatenate(
        [x, jnp.ones((n, 1), jnp.float32), jnp.zeros((n, DA - d_in - 1), jnp.float32)],
        axis=1,
    )
    wa = jnp.concatenate(
        [W, b[:, None], jnp.zeros((d_out, DA - d_in - 1), jnp.float32)], axis=1
    )

    info = plsc.get_sparse_core_info()
    nw = info.num_cores * info.num_subcores
    kpw = -(-e // (nw * CH))
    e_pad = kpw * nw * CH
    trash = n
    colp = jnp.concatenate([col, jnp.zeros((e_pad - e,), jnp.int32)])
    rowp = jnp.concatenate([row, jnp.full((e_pad - e,), trash, jnp.int32)])

    parts = _sc_aggregate(xa, colp, rowp, kpw)
    p0 = parts[0, :n]
    p1 = parts[1, :n]
    return _tc_finish(xa, p0, p1, wa, gamma.reshape(1, d_out), beta.reshape(1, d_out))
